# baseline (speedup 1.0000x reference)
import jax
import jax.numpy as jnp
from jax import lax
from jax.experimental import pallas as pl
from jax.experimental.pallas import tpu as pltpu

BATCH = 524288
DIM = 32
LABELS = 8192
BLKC = 131072
GRID = BATCH // BLKC
ACCW = 1024
SLICES = BLKC // ACCW

_DN_LANE = (((1,), (0,)), ((), ()))
_DN_LAST = (((1,), (1,)), ((), ()))


def _fused_tc(x_ref, y_ref, o_ref, acc_ref):
    i = pl.program_id(0)

    @pl.when(i == 0)
    def _():
        acc_ref[...] = jnp.zeros_like(acc_ref)

    a = acc_ref[...]
    x = x_ref[...]
    for k in range(SLICES):
        a += x[:, k * ACCW:(k + 1) * ACCW]
    acc_ref[...] = a

    @pl.when(i == GRID - 1)
    def _():
        acc = acc_ref[...]
        sw = lax.dot_general(
            acc, jnp.ones((ACCW, 8), jnp.float32), _DN_LANE,
            preferred_element_type=jnp.float32,
        )
        s8 = 0.125 * lax.dot_general(
            jnp.ones((8, 8), jnp.float32), sw, _DN_LAST,
            preferred_element_type=jnp.float32,
        )
        yt = y_ref[...]
        m8 = lax.dot_general(s8, yt, _DN_LANE, preferred_element_type=jnp.float32)
        q8 = lax.dot_general(
            jnp.ones((8, DIM), jnp.float32), yt * yt, _DN_LANE,
            preferred_element_type=jnp.float32,
        )
        m = m8[0:1, :]
        q = q8[0:1, :]
        metric = jnp.sign(m) * (m * m) / q
        maxv = jnp.max(metric)
        col = lax.broadcasted_iota(jnp.int32, metric.shape, 1)
        cand = jnp.where(metric == maxv, col, 2**30)
        o_ref[0, 0] = jnp.min(cand)


def kernel(inputs, mean_distances):
    xt = inputs.T
    yt = mean_distances.T
    idx = pl.pallas_call(
        _fused_tc,
        grid=(GRID,),
        in_specs=[
            pl.BlockSpec((DIM, BLKC), lambda i: (0, i)),
            pl.BlockSpec((DIM, LABELS), lambda i: (0, 0)),
        ],
        out_specs=pl.BlockSpec(memory_space=pltpu.SMEM),
        out_shape=jax.ShapeDtypeStruct((1, 1), jnp.int32),
        scratch_shapes=[pltpu.VMEM((DIM, ACCW), jnp.float32)],
    )(xt, yt)
    return idx.reshape(1)


# speedup vs baseline: 1.0538x; 1.0538x over previous
import jax
import jax.numpy as jnp
from jax import lax
from jax.experimental import pallas as pl
from jax.experimental.pallas import tpu as pltpu

BATCH = 524288
DIM = 32
LABELS = 8192
BLKC = 65536
GRID = BATCH // BLKC
ACCW = 1024
SLICES = BLKC // ACCW

_DN_LANE = (((1,), (0,)), ((), ()))
_DN_LAST = (((1,), (1,)), ((), ()))


def _fused_tc(x_ref, y_ref, o_ref, acc_ref, q_ref):
    i = pl.program_id(0)

    @pl.when(i == 0)
    def _():
        acc_ref[...] = jnp.zeros_like(acc_ref)
        yt0 = y_ref[...]
        q_ref[...] = lax.dot_general(
            jnp.ones((8, DIM), jnp.float32), yt0 * yt0, _DN_LANE,
            preferred_element_type=jnp.float32,
        )

    a = acc_ref[...]
    x = x_ref[...]
    for k in range(SLICES):
        a += x[:, k * ACCW:(k + 1) * ACCW]
    acc_ref[...] = a

    @pl.when(i == GRID - 1)
    def _():
        acc = acc_ref[...]
        sw = lax.dot_general(
            acc, jnp.ones((ACCW, 8), jnp.float32), _DN_LANE,
            preferred_element_type=jnp.float32,
        )
        s8 = 0.125 * lax.dot_general(
            jnp.ones((8, 8), jnp.float32), sw, _DN_LAST,
            preferred_element_type=jnp.float32,
        )
        yt = y_ref[...]
        m8 = lax.dot_general(s8, yt, _DN_LANE, preferred_element_type=jnp.float32)
        m = m8[0:1, :]
        q = q_ref[0:1, :]
        metric = jnp.sign(m) * (m * m) / q
        maxv = jnp.max(metric)
        col = lax.broadcasted_iota(jnp.int32, metric.shape, 1)
        cand = jnp.where(metric == maxv, col, 2**30)
        o_ref[0, 0] = jnp.min(cand)


def kernel(inputs, mean_distances):
    xt = inputs.T
    yt = mean_distances.T
    idx = pl.pallas_call(
        _fused_tc,
        grid=(GRID,),
        in_specs=[
            pl.BlockSpec((DIM, BLKC), lambda i: (0, i)),
            pl.BlockSpec((DIM, LABELS), lambda i: (0, 0)),
        ],
        out_specs=pl.BlockSpec(memory_space=pltpu.SMEM),
        out_shape=jax.ShapeDtypeStruct((1, 1), jnp.int32),
        scratch_shapes=[pltpu.VMEM((DIM, ACCW), jnp.float32),
                        pltpu.VMEM((8, LABELS), jnp.float32)],
    )(xt, yt)
    return idx.reshape(1)
